# baseline (speedup 1.0000x reference)
import functools

import jax
import jax.numpy as jnp
from jax import lax
from jax.experimental import pallas as pl
from jax.experimental.pallas import tpu as pltpu
from jax.experimental.pallas import tpu_sc as plsc

_VOCAB = 100000
_DIM = 64
_B = 1024
_L = 20
_NTOK = _B * _L
_VT = 2048
_NBUF = 4
_GRID = (_VOCAB + _VT - 1) // _VT
_LAST = _VOCAB - (_GRID - 1) * _VT
_VPAD = _GRID * _VT
_MAX_NORM = 1.0


@functools.lru_cache(maxsize=None)
def _make_sc_gather():
    info = plsc.get_sparse_core_info()
    nc, ns = info.num_cores, info.num_subcores
    nw = nc * ns
    bpw = _NTOK // nw
    assert _NTOK % nw == 0 and bpw % 8 == 0
    mesh = plsc.VectorSubcoreMesh(core_axis_name="c", subcore_axis_name="s")

    @functools.partial(
        pl.kernel,
        mesh=mesh,
        out_type=jax.ShapeDtypeStruct((_NTOK, _DIM), jnp.float32),
        scratch_types=[
            pltpu.VMEM((bpw,), jnp.int32),
            pltpu.VMEM((bpw, _DIM), jnp.float32),
            pltpu.SemaphoreType.DMA,
        ],
        compiler_params=pltpu.CompilerParams(use_tc_tiling_on_sc=False),
    )
    def gather_k(table_hbm, idx_hbm, out_hbm, idx_v, rows_v, sem):
        wid = lax.axis_index("s") * nc + lax.axis_index("c")
        base = wid * bpw
        pltpu.sync_copy(idx_hbm.at[pl.ds(base, bpw)], idx_v)
        pltpu.async_copy(table_hbm.at[idx_v], rows_v, sem).wait()
        pltpu.sync_copy(rows_v, out_hbm.at[pl.ds(base, bpw)])

    return gather_k


def _pool_body(g_ref, x_ref):
    g = g_ref[...]
    ss = jnp.sum(g * g, axis=-1, keepdims=True)
    norm = jnp.sqrt(ss)
    scale = jnp.minimum(1.0, _MAX_NORM / jnp.maximum(norm, 1e-7))
    x_ref[...] = jnp.sum(g * scale, axis=0)


@functools.lru_cache(maxsize=None)
def _make_pool():
    return pl.pallas_call(
        _pool_body,
        out_shape=jax.ShapeDtypeStruct((_B, _DIM), jnp.float32),
    )


def _out_copy(obuf, obuf_last, o_hbm, sems, j):
    if j < _GRID - 1:
        return pltpu.make_async_copy(
            obuf.at[j % _NBUF],
            o_hbm.at[:, pl.ds(j * _VT, _VT)],
            sems.at[j % _NBUF],
        )
    return pltpu.make_async_copy(
        obuf_last,
        o_hbm.at[:, pl.ds(j * _VT, _LAST)],
        sems.at[_NBUF],
    )


def _mm_body(x_ref, w_ref, b_ref, o_hbm, obuf, obuf_last, sems):
    i = pl.program_id(0)
    slot = lax.rem(i, _NBUF)

    @pl.when(i >= _NBUF)
    def _():
        pltpu.make_async_copy(
            obuf.at[slot],
            o_hbm.at[:, pl.ds((i - _NBUF) * _VT, _VT)],
            sems.at[slot],
        ).wait()

    @pl.when(i < _GRID - 1)
    def _():
        obuf[slot] = (
            lax.dot_general(
                x_ref[...], w_ref[...],
                (((1,), (1,)), ((), ())),
                preferred_element_type=jnp.float32,
            )
            + b_ref[:, pl.ds(i * _VT, _VT)]
        )
        pltpu.make_async_copy(
            obuf.at[slot],
            o_hbm.at[:, pl.ds(i * _VT, _VT)],
            sems.at[slot],
        ).start()

    @pl.when(i == _GRID - 1)
    def _():
        obuf_last[...] = (
            lax.dot_general(
                x_ref[...], w_ref[pl.ds(0, _LAST), :],
                (((1,), (1,)), ((), ())),
                preferred_element_type=jnp.float32,
            )
            + b_ref[:, pl.ds(i * _VT, _LAST)]
        )
        _out_copy(obuf, obuf_last, o_hbm, sems, _GRID - 1).start()
        for j in range(_GRID - _NBUF, _GRID):
            _out_copy(obuf, obuf_last, o_hbm, sems, j).wait()


@functools.lru_cache(maxsize=None)
def _make_mm():
    return pl.pallas_call(
        _mm_body,
        grid=(_GRID,),
        in_specs=[
            pl.BlockSpec((_B, _DIM), lambda i: (0, 0)),
            pl.BlockSpec((_VT, _DIM), lambda i: (i, 0)),
            pl.BlockSpec((1, _VPAD), lambda i: (0, 0)),
        ],
        out_specs=pl.BlockSpec(memory_space=pl.ANY),
        out_shape=jax.ShapeDtypeStruct((_B, _VOCAB), jnp.float32),
        scratch_shapes=[
            pltpu.VMEM((_NBUF, _B, _VT), jnp.float32),
            pltpu.VMEM((_B, _LAST), jnp.float32),
            pltpu.SemaphoreType.DMA((_NBUF + 1,)),
        ],
    )


def kernel(inputs_, embed_table, W, b):
    idx = inputs_.T.reshape(-1)
    gathered = _make_sc_gather()(embed_table, idx)
    g3 = gathered.reshape(_L, _B, _DIM)
    x = _make_pool()(g3)
    b_pad = jnp.pad(b, (0, _VPAD - _VOCAB)).reshape(1, _VPAD)
    return _make_mm()(x, W, b_pad)


# speedup vs baseline: 1.0026x; 1.0026x over previous
import functools

import jax
import jax.numpy as jnp
from jax import lax
from jax.experimental import pallas as pl
from jax.experimental.pallas import tpu as pltpu
from jax.experimental.pallas import tpu_sc as plsc

_VOCAB = 100000
_DIM = 64
_B = 1024
_L = 20
_NTOK = _B * _L
_VT = 2048
_NBUF = 4
_GRID = (_VOCAB + _VT - 1) // _VT
_LAST = _VOCAB - (_GRID - 1) * _VT
_VPAD = _GRID * _VT
_MAX_NORM = 1.0


@functools.lru_cache(maxsize=None)
def _make_sc_gather():
    info = plsc.get_sparse_core_info()
    nc, ns = info.num_cores, info.num_subcores
    nw = nc * ns
    bpw = _NTOK // nw
    assert _NTOK % nw == 0 and bpw % 8 == 0
    mesh = plsc.VectorSubcoreMesh(core_axis_name="c", subcore_axis_name="s")

    @functools.partial(
        pl.kernel,
        mesh=mesh,
        out_type=jax.ShapeDtypeStruct((_NTOK, _DIM), jnp.float32),
        scratch_types=[
            pltpu.VMEM((bpw,), jnp.int32),
            pltpu.VMEM((bpw, _DIM), jnp.float32),
            pltpu.SemaphoreType.DMA,
        ],
        compiler_params=pltpu.CompilerParams(use_tc_tiling_on_sc=False),
    )
    def gather_k(table_hbm, idx_hbm, out_hbm, idx_v, rows_v, sem):
        wid = lax.axis_index("s") * nc + lax.axis_index("c")
        base = wid * bpw
        pltpu.sync_copy(idx_hbm.at[pl.ds(base, bpw)], idx_v)
        pltpu.async_copy(table_hbm.at[idx_v], rows_v, sem).wait()
        pltpu.sync_copy(rows_v, out_hbm.at[pl.ds(base, bpw)])

    return gather_k


def _out_copy(obuf, obuf_last, o_hbm, sems, j):
    if j < _GRID - 1:
        return pltpu.make_async_copy(
            obuf.at[j % _NBUF],
            o_hbm.at[:, pl.ds(j * _VT, _VT)],
            sems.at[j % _NBUF],
        )
    return pltpu.make_async_copy(
        obuf_last,
        o_hbm.at[:, pl.ds(j * _VT, _LAST)],
        sems.at[_NBUF],
    )


def _mm_body(g_ref, w_ref, b_ref, o_hbm, x_ref, obuf, obuf_last, sems):
    i = pl.program_id(0)
    slot = lax.rem(i, _NBUF)

    @pl.when(i == 0)
    def _():
        g = g_ref[...]
        ss = jnp.sum(g * g, axis=-1, keepdims=True)
        norm = jnp.sqrt(ss)
        scale = jnp.minimum(1.0, _MAX_NORM / jnp.maximum(norm, 1e-7))
        x_ref[...] = jnp.sum(g * scale, axis=0)

    @pl.when(i >= _NBUF)
    def _():
        pltpu.make_async_copy(
            obuf.at[slot],
            o_hbm.at[:, pl.ds((i - _NBUF) * _VT, _VT)],
            sems.at[slot],
        ).wait()

    @pl.when(i < _GRID - 1)
    def _():
        obuf[slot] = (
            lax.dot_general(
                x_ref[...], w_ref[...],
                (((1,), (1,)), ((), ())),
                preferred_element_type=jnp.float32,
            )
            + b_ref[:, pl.ds(i * _VT, _VT)]
        )
        pltpu.make_async_copy(
            obuf.at[slot],
            o_hbm.at[:, pl.ds(i * _VT, _VT)],
            sems.at[slot],
        ).start()

    @pl.when(i == _GRID - 1)
    def _():
        obuf_last[...] = (
            lax.dot_general(
                x_ref[...], w_ref[pl.ds(0, _LAST), :],
                (((1,), (1,)), ((), ())),
                preferred_element_type=jnp.float32,
            )
            + b_ref[:, pl.ds(i * _VT, _LAST)]
        )
        _out_copy(obuf, obuf_last, o_hbm, sems, _GRID - 1).start()
        for j in range(_GRID - _NBUF, _GRID):
            _out_copy(obuf, obuf_last, o_hbm, sems, j).wait()


@functools.lru_cache(maxsize=None)
def _make_mm():
    return pl.pallas_call(
        _mm_body,
        grid=(_GRID,),
        in_specs=[
            pl.BlockSpec((_L, _B, _DIM), lambda i: (0, 0, 0)),
            pl.BlockSpec((_VT, _DIM), lambda i: (i, 0)),
            pl.BlockSpec((1, _VPAD), lambda i: (0, 0)),
        ],
        out_specs=pl.BlockSpec(memory_space=pl.ANY),
        out_shape=jax.ShapeDtypeStruct((_B, _VOCAB), jnp.float32),
        scratch_shapes=[
            pltpu.VMEM((_B, _DIM), jnp.float32),
            pltpu.VMEM((_NBUF, _B, _VT), jnp.float32),
            pltpu.VMEM((_B, _LAST), jnp.float32),
            pltpu.SemaphoreType.DMA((_NBUF + 1,)),
        ],
    )


def kernel(inputs_, embed_table, W, b):
    idx = inputs_.T.reshape(-1)
    gathered = _make_sc_gather()(embed_table, idx)
    g3 = gathered.reshape(_L, _B, _DIM)
    b_pad = jnp.pad(b, (0, _VPAD - _VOCAB)).reshape(1, _VPAD)
    return _make_mm()(g3, W, b_pad)
